# Initial kernel scaffold; baseline (speedup 1.0000x reference)
import jax
import jax.numpy as jnp
from jax.experimental import pallas as pl


def kernel(user_indices, item_indices, user_emb, item_emb, edge_index, edge_weight):
    raise NotImplementedError("write your pallas kernel here")


# speedup vs baseline: 24.2985x; 24.2985x over previous
import functools

import jax
import jax.numpy as jnp
from jax import lax
from jax.experimental import pallas as pl
from jax.experimental.pallas import tpu as pltpu
from jax.experimental.pallas import tpu_sc as plsc

NUM_USERS = 50000
NUM_ITEMS = 50000
NUM_NODES = NUM_USERS + NUM_ITEMS
DIM = 16
NUM_EDGES = 3200000
BATCH = 16384
NUM_LAYERS = 3

NC = 2
NS = 16
SUB = 128
CHUNK = 16 * SUB
STEPS = 98
EDGES_PAD = NS * STEPS * CHUNK
EROWS = EDGES_PAD // SUB

HALF = NUM_USERS
YROWS = 51200
JUNK_MASK = 1023

_mesh = plsc.VectorSubcoreMesh(core_axis_name="c", subcore_axis_name="s")


@functools.partial(
    pl.kernel,
    out_type=jax.ShapeDtypeStruct((NUM_NODES, DIM), jnp.float32),
    mesh=_mesh,
    compiler_params=pltpu.CompilerParams(use_tc_tiling_on_sc=False, needs_layout_passes=False),
    scratch_types=[
        pltpu.VMEM_SHARED((YROWS, DIM), jnp.float32),
        pltpu.VMEM((16, SUB), jnp.int32),
        pltpu.VMEM((16, SUB), jnp.int32),
        pltpu.VMEM((CHUNK,), jnp.float32),
        pltpu.VMEM((CHUNK, DIM), jnp.float32),
        pltpu.SemaphoreType.DMA,
    ],
)
def _layer(x_hbm, src_hbm, dst_hbm, w_hbm, out_hbm,
           y_sh, src_v, dst_v, w_v, rows_v, sem):
    c = lax.axis_index("c")
    s = lax.axis_index("s")
    lo = c * HALF

    def _z(i, _):
        rows_v[i] = jnp.zeros((DIM,), jnp.float32)
        return 0
    lax.fori_loop(0, SUB, _z, 0)

    def _zcp(k, _):
        pltpu.sync_copy(rows_v.at[pl.ds(0, SUB)],
                        y_sh.at[pl.ds(s * (YROWS // NS) + k * SUB, SUB)])
        return 0
    lax.fori_loop(0, (YROWS // NS) // SUB, _zcp, 0)
    plsc.subcore_barrier()

    erow0 = s * STEPS * 16

    def _step(i, _):
        row0 = erow0 + i * 16
        pltpu.sync_copy(src_hbm.at[pl.ds(row0, 16)], src_v)
        pltpu.sync_copy(dst_hbm.at[pl.ds(row0, 16)], dst_v)
        pltpu.sync_copy(w_hbm.at[pl.ds(row0 * SUB, CHUNK)], w_v)

        gats = [
            pltpu.async_copy(x_hbm.at[src_v.at[b]],
                             rows_v.at[pl.ds(b * SUB, SUB)], sem)
            for b in range(16)
        ]

        def _loc(r, _):
            for cc in range(8):
                sl = pl.ds(cc * 16, 16)
                d16 = dst_v[r, sl]
                loc = d16 - lo
                ok = (loc >= 0) & (loc < HALF)
                junk = HALF + (d16 & JUNK_MASK)
                dst_v[r, sl] = jnp.where(ok, loc, junk)
            return 0
        lax.fori_loop(0, 16, _loc, 0)

        for g in gats:
            g.wait()

        def _scale(g, _):
            w16 = w_v[pl.ds(g * 16, 16)]
            for k in range(16):
                j = g * 16 + k
                rows_v[j] = rows_v[j] * jnp.broadcast_to(w16[k], (DIM,))
            return 0
        lax.fori_loop(0, CHUNK // 16, _scale, 0)

        scats = [
            pltpu.async_copy(rows_v.at[pl.ds(b * SUB, SUB)],
                             y_sh.at[dst_v.at[b]], sem, add=True)
            for b in range(16)
        ]
        for g in scats:
            g.wait()
        return 0

    lax.fori_loop(0, STEPS, _step, 0)
    plsc.subcore_barrier()

    WB = 200
    NBLK = HALF // WB

    def _wb(k, _):
        blk = k * NS + s

        @pl.when(blk < NBLK)
        def _():
            r0 = blk * WB
            pltpu.sync_copy(y_sh.at[pl.ds(r0, WB)], rows_v.at[pl.ds(0, WB)])
            pltpu.sync_copy(rows_v.at[pl.ds(0, WB)],
                            out_hbm.at[pl.ds(lo + r0, WB)])
        return 0
    lax.fori_loop(0, (NBLK + NS - 1) // NS, _wb, 0)


PW = BATCH // (NC * NS)
PROWS = PW // SUB


@functools.partial(
    pl.kernel,
    out_type=jax.ShapeDtypeStruct((BATCH,), jnp.float32),
    mesh=_mesh,
    compiler_params=pltpu.CompilerParams(use_tc_tiling_on_sc=False, needs_layout_passes=False),
    scratch_types=[
        pltpu.VMEM((PROWS, SUB), jnp.int32),
        pltpu.VMEM((PROWS, SUB), jnp.int32),
        pltpu.VMEM((PW, DIM), jnp.float32),
        pltpu.VMEM((PW, DIM), jnp.float32),
        pltpu.VMEM((PW,), jnp.float32),
        pltpu.SemaphoreType.DMA,
    ],
)
def _score(x_hbm, ui_hbm, ii_hbm, out_hbm,
           ui_v, ii_v, ur_v, ir_v, sc_v, sem):
    c = lax.axis_index("c")
    s = lax.axis_index("s")
    wid = s * NC + c

    for r in range(PROWS):
        pltpu.sync_copy(ui_hbm.at[pl.ds(wid * PW + r * SUB, SUB)], ui_v.at[r])
        pltpu.sync_copy(ii_hbm.at[pl.ds(wid * PW + r * SUB, SUB)], ii_v.at[r])

    def _off(r, _):
        for cc in range(8):
            sl = pl.ds(cc * 16, 16)
            ii_v[r, sl] = ii_v[r, sl] + NUM_USERS
        return 0
    lax.fori_loop(0, PROWS, _off, 0)

    cps = []
    for r in range(PROWS):
        cps.append(pltpu.async_copy(x_hbm.at[ui_v.at[r]],
                                    ur_v.at[pl.ds(r * SUB, SUB)], sem))
        cps.append(pltpu.async_copy(x_hbm.at[ii_v.at[r]],
                                    ir_v.at[pl.ds(r * SUB, SUB)], sem))
    for g in cps:
        g.wait()

    def _dot(g, _):
        pid = g * 16 + lax.iota(jnp.int32, 16)
        acc = jnp.zeros((16,), jnp.float32)
        for d in range(DIM):
            dd = jnp.full((16,), d, jnp.int32)
            u = plsc.load_gather(ur_v, [pid, dd])
            it = plsc.load_gather(ir_v, [pid, dd])
            acc = acc + u * it
        sc_v[pl.ds(g * 16, 16)] = acc
        return 0
    lax.fori_loop(0, PW // 16, _dot, 0)

    pltpu.sync_copy(sc_v, out_hbm.at[pl.ds(wid * PW, PW)])


def kernel(user_indices, item_indices, user_emb, item_emb, edge_index, edge_weight):
    x = jnp.concatenate([user_emb, item_emb], axis=0)

    pad = EDGES_PAD - NUM_EDGES
    src = jnp.concatenate([edge_index[0].astype(jnp.int32),
                           jnp.zeros((pad,), jnp.int32)])
    dst = jnp.concatenate([edge_index[1].astype(jnp.int32),
                           jnp.zeros((pad,), jnp.int32)])
    w = jnp.concatenate([edge_weight,
                         jnp.zeros((pad,), jnp.float32)])
    src2d = src.reshape(EROWS, SUB)
    dst2d = dst.reshape(EROWS, SUB)

    for _ in range(NUM_LAYERS):
        x = _layer(x, src2d, dst2d, w)

    return _score(x, user_indices.astype(jnp.int32),
                  item_indices.astype(jnp.int32))
